# baseline (speedup 1.0000x reference)
import functools

import jax
import jax.numpy as jnp
import numpy as np
from jax import lax
from jax.experimental import pallas as pl
from jax.experimental.pallas import tpu as pltpu
from jax.experimental.pallas import tpu_sc as plsc

N = 10000
E = 320000
H = 8
D = 16
HD = H * D
ROW = HD + 16
SEM = 128
NC, NS, L = 2, 16, 16
NW = NC * NS
CHUNK = 128
NCHUNK = E // CHUNK
NPAD = 10112
NPT = NPAD // NS
BN = 1000
NB = N // BN

_HIGH = lax.Precision.HIGHEST


def _prep_body(x_ref, wt_ref, wr_ref, t_ref, r_ref, m_ref):
    i = pl.program_id(0)
    xb = x_ref[...]
    t = jnp.dot(xb, wt_ref[...], preferred_element_type=jnp.float32,
                precision=_HIGH)
    r = jnp.dot(xb, wr_ref[...], preferred_element_type=jnp.float32,
                precision=_HIGH)
    t_ref[...] = t
    r_ref[...] = r
    elmax = jnp.max(t[:, HD:ROW], axis=0, keepdims=True)
    ermax = jnp.max(r, axis=0, keepdims=True)
    cur = jnp.concatenate([elmax, ermax], axis=0)

    @pl.when(i == 0)
    def _():
        m_ref[...] = cur

    @pl.when(i != 0)
    def _():
        m_ref[...] = jnp.maximum(m_ref[...], cur)


_prep_call = pl.pallas_call(
    _prep_body,
    grid=(NB,),
    in_specs=[
        pl.BlockSpec((BN, HD), lambda i: (i, 0)),
        pl.BlockSpec((HD, ROW), lambda i: (0, 0)),
        pl.BlockSpec((HD, L), lambda i: (0, 0)),
    ],
    out_specs=[
        pl.BlockSpec((BN, ROW), lambda i: (i, 0)),
        pl.BlockSpec((BN, L), lambda i: (i, 0)),
        pl.BlockSpec((2, L), lambda i: (0, 0)),
    ],
    out_shape=[
        jax.ShapeDtypeStruct((N, ROW), jnp.float32),
        jax.ShapeDtypeStruct((N, L), jnp.float32),
        jax.ShapeDtypeStruct((2, L), jnp.float32),
    ],
)


def _sc_body(t0, r0, s0, d0, m0, t1, r1, s1, d1, m1, zeros_hbm, out_hbm,
             idx_s, idx_df, idx_d2, g1, g2, ob, mv, acc_ref, sem):
    cid = lax.axis_index("c")
    sid = lax.axis_index("s")
    wid = sid * NC + cid

    def _bcast(v, h):
        idx = jnp.full((L,), h, jnp.int32)
        return lax.gather(
            v, idx[:, None],
            lax.GatherDimensionNumbers(offset_dims=(), collapsed_slice_dims=(0,),
                                       start_index_map=(0,)),
            (1,), mode=lax.GatherScatterMode.PROMISE_IN_BOUNDS)

    for p, (T, R, S, Dd, M) in enumerate(((t0, r0, s0, d0, m0),
                                          (t1, r1, s1, d1, m1))):
        pltpu.sync_copy(zeros_hbm, acc_ref.at[pl.ds(sid * NPT, NPT)])
        pltpu.sync_copy(M, mv)
        msum = mv[0, :] + mv[1, :]
        cvec = jnp.where(msum > 0, msum, 0.2 * msum)
        plsc.subcore_barrier()

        nloc = (NCHUNK // NW) + jnp.where(wid < (NCHUNK % NW), 1, 0)

        def chunk_body(j, _):
            k = wid + j * NW
            off = k * CHUNK
            pltpu.sync_copy(S.at[pl.ds(off, CHUNK)], idx_s)
            pltpu.sync_copy(Dd.at[pl.ds(off, CHUNK)], idx_df)
            pltpu.sync_copy(Dd.at[pl.ds(off, CHUNK)], idx_d2.at[0])
            pltpu.async_copy(T.at[idx_s], g1, sem).wait()
            pltpu.async_copy(R.at[idx_df], g2, sem).wait()

            @plsc.parallel_loop(0, CHUNK, unroll=2)
            def _(i):
                el = g1[i, pl.ds(HD, L)]
                er = g2[i, :]
                sv = el + er
                ev = jnp.where(sv > 0, sv, 0.2 * sv)
                ee = jnp.exp(ev - cvec)
                ob[i, pl.ds(HD, L)] = ee
                for h in range(H):
                    ob[i, pl.ds(h * D, D)] = g1[i, pl.ds(h * D, D)] * ee[h]
            pltpu.sync_copy(ob, acc_ref.at[idx_d2.at[0]], add=True)
            return 0

        lax.fori_loop(0, nloc, chunk_body, 0)
        plsc.subcore_barrier()
        pltpu.sync_copy(
            acc_ref.at[pl.ds(sid * NPT, NPT)],
            out_hbm.at[p, cid, pl.ds(sid * NPT, NPT)])
        plsc.subcore_barrier()


def _make_sc_call():
    mesh = plsc.VectorSubcoreMesh(core_axis_name="c", subcore_axis_name="s",
                                  num_cores=NC, num_subcores=NS)

    return pl.kernel(
        _sc_body,
        out_type=jax.ShapeDtypeStruct((2, NC, NPAD, ROW), jnp.float32),
        mesh=mesh,
        scratch_types=[
            pltpu.VMEM((CHUNK,), jnp.int32),
            pltpu.VMEM((CHUNK,), jnp.int32),
            pltpu.VMEM((1, CHUNK), jnp.int32),
            pltpu.VMEM((CHUNK, ROW), jnp.float32),
            pltpu.VMEM((CHUNK, L), jnp.float32),
            pltpu.VMEM((CHUNK, ROW), jnp.float32),
            pltpu.VMEM((2, L), jnp.float32),
            pltpu.VMEM_SHARED((NPAD, ROW), jnp.float32),
            pltpu.SemaphoreType.DMA,
        ],
        compiler_params=pltpu.CompilerParams(use_tc_tiling_on_sc=False),
    )


_sc_call = _make_sc_call()


def _finish_body(p_ref, b_ref, k_ref, sw_ref, sb_ref, sv_ref,
                 z0_ref, z1_ref, w_ref):
    i = pl.program_id(0)
    wps = []
    for pi, zref in ((0, z0_ref), (1, z1_ref)):
        agg = p_ref[pi, 0] + p_ref[pi, 1]
        den = jnp.dot(agg[:, HD:ROW], k_ref[...],
                      preferred_element_type=jnp.float32, precision=_HIGH)
        q = agg[:, :HD] / jnp.maximum(den, 1e-9) + b_ref[pi]
        z = jnp.where(q > 0, q, jnp.exp(jnp.minimum(q, 0.0)) - 1.0)
        zref[...] = z
        t = jnp.tanh(jnp.dot(z, sw_ref[...], preferred_element_type=jnp.float32,
                             precision=_HIGH) + sb_ref[...])
        wps.append(jnp.full((1, 1, HD), jnp.sum(t * sv_ref[...])))
    wp = jnp.concatenate(wps, axis=0)

    @pl.when(i == 0)
    def _():
        w_ref[...] = wp

    @pl.when(i != 0)
    def _():
        w_ref[...] = w_ref[...] + wp


_finish_call = pl.pallas_call(
    _finish_body,
    grid=(NB,),
    in_specs=[
        pl.BlockSpec((2, NC, BN, ROW), lambda i: (0, 0, i, 0)),
        pl.BlockSpec((2, 1, HD), lambda i: (0, 0, 0)),
        pl.BlockSpec((L, HD), lambda i: (0, 0)),
        pl.BlockSpec((HD, SEM), lambda i: (0, 0)),
        pl.BlockSpec((1, SEM), lambda i: (0, 0)),
        pl.BlockSpec((1, SEM), lambda i: (0, 0)),
    ],
    out_specs=[
        pl.BlockSpec((BN, HD), lambda i: (i, 0)),
        pl.BlockSpec((BN, HD), lambda i: (i, 0)),
        pl.BlockSpec((2, 1, HD), lambda i: (0, 0, 0)),
    ],
    out_shape=[
        jax.ShapeDtypeStruct((N, HD), jnp.float32),
        jax.ShapeDtypeStruct((N, HD), jnp.float32),
        jax.ShapeDtypeStruct((2, 1, HD), jnp.float32),
    ],
)


def _combine_body(z0_ref, z1_ref, w_ref, o_ref):
    w0 = w_ref[0, 0, 0] / np.float32(N)
    w1 = w_ref[1, 0, 0] / np.float32(N)
    m = jnp.maximum(w0, w1)
    e0 = jnp.exp(w0 - m)
    e1 = jnp.exp(w1 - m)
    b0 = e0 / (e0 + e1)
    b1 = e1 / (e0 + e1)
    o_ref[...] = b0 * z0_ref[...] + b1 * z1_ref[...]


_combine_call = pl.pallas_call(
    _combine_body,
    grid=(NB,),
    in_specs=[
        pl.BlockSpec((BN, HD), lambda i: (i, 0)),
        pl.BlockSpec((BN, HD), lambda i: (i, 0)),
        pl.BlockSpec((2, 1, HD), lambda i: (0, 0, 0)),
    ],
    out_specs=pl.BlockSpec((BN, HD), lambda i: (i, 0)),
    out_shape=jax.ShapeDtypeStruct((N, HD), jnp.float32),
)


def _pack_weights(W, al, ar):
    sel = jnp.asarray(np.kron(np.eye(H), np.ones((D, 1))), jnp.float32)
    Wl = (W * al.reshape(-1)[None, :]) @ sel
    Wr = (W * ar.reshape(-1)[None, :]) @ sel
    Wt = jnp.concatenate([W, Wl, Wl], axis=1)
    Wr2 = jnp.concatenate([Wr, Wr], axis=1)
    return Wt, Wr2


def kernel(x, edge_index_0, edge_index_1, W_gat0, b_gat0, attn_l0, attn_r0,
           W_gat1, b_gat1, attn_l1, attn_r1, sem_W, sem_b, sem_v):
    Wt0, Wr20 = _pack_weights(W_gat0, attn_l0, attn_r0)
    Wt1, Wr21 = _pack_weights(W_gat1, attn_l1, attn_r1)
    T0, R0, m0 = _prep_call(x, Wt0, Wr20)
    T1, R1, m1 = _prep_call(x, Wt1, Wr21)
    zeros = jnp.zeros((NPT, ROW), jnp.float32)
    P = _sc_call(T0, R0, edge_index_0[0], edge_index_0[1], m0,
                 T1, R1, edge_index_1[0], edge_index_1[1], m1, zeros)
    ksel = jnp.asarray(np.kron(np.eye(H), np.ones((1, D))), jnp.float32)
    K = jnp.concatenate([ksel, jnp.zeros((H, HD), jnp.float32)], axis=0)
    bstack = jnp.stack([b_gat0, b_gat1]).reshape(2, 1, HD)
    z0, z1, wsum = _finish_call(P, bstack, K, sem_W, sem_b.reshape(1, SEM),
                                sem_v.reshape(SEM, 1).T)
    return _combine_call(z0, z1, wsum)


# speedup vs baseline: 1.8260x; 1.8260x over previous
import functools

import jax
import jax.numpy as jnp
import numpy as np
from jax import lax
from jax.experimental import pallas as pl
from jax.experimental.pallas import tpu as pltpu
from jax.experimental.pallas import tpu_sc as plsc

N = 10000
E = 320000
H = 8
D = 16
HD = H * D
ROW = HD + 16
SEM = 128
NC, NS, L = 2, 16, 16
NW = NC * NS
CHUNK = 80
NCHUNK = E // CHUNK
NLOC = NCHUNK // NW
NPAD = 10112
NPT = NPAD // NS
BN = 1000
NB = N // BN

_HIGH = lax.Precision.HIGHEST


def _prep_body(x_ref, wt_ref, wr_ref, t_ref, r_ref, m_ref):
    i = pl.program_id(0)
    xb = x_ref[...]
    t = jnp.dot(xb, wt_ref[...], preferred_element_type=jnp.float32,
                precision=_HIGH)
    r = jnp.dot(xb, wr_ref[...], preferred_element_type=jnp.float32,
                precision=_HIGH)
    t_ref[...] = t
    r_ref[...] = r
    elmax = jnp.max(t[:, HD:ROW], axis=0, keepdims=True)
    ermax = jnp.max(r, axis=0, keepdims=True)
    cur = jnp.concatenate([elmax, ermax], axis=0)

    @pl.when(i == 0)
    def _():
        m_ref[...] = cur

    @pl.when(i != 0)
    def _():
        m_ref[...] = jnp.maximum(m_ref[...], cur)


_prep_call = pl.pallas_call(
    _prep_body,
    grid=(NB,),
    in_specs=[
        pl.BlockSpec((BN, HD), lambda i: (i, 0)),
        pl.BlockSpec((HD, ROW), lambda i: (0, 0)),
        pl.BlockSpec((HD, L), lambda i: (0, 0)),
    ],
    out_specs=[
        pl.BlockSpec((BN, ROW), lambda i: (i, 0)),
        pl.BlockSpec((BN, L), lambda i: (i, 0)),
        pl.BlockSpec((2, L), lambda i: (0, 0)),
    ],
    out_shape=[
        jax.ShapeDtypeStruct((N, ROW), jnp.float32),
        jax.ShapeDtypeStruct((N, L), jnp.float32),
        jax.ShapeDtypeStruct((2, L), jnp.float32),
    ],
)


def _sc_body(t0, r0, s0, d0, m0, t1, r1, s1, d1, m1, zeros_hbm, out_hbm,
             eibs, eibd, g1, g2, ob, mv, acc_ref, semi, semg, sems):
    cid = lax.axis_index("c")
    sid = lax.axis_index("s")
    wid = sid * NC + cid

    for p, (T, R, S1, D1, M) in enumerate(((t0, r0, s0, d0, m0),
                                           (t1, r1, s1, d1, m1))):
        pltpu.sync_copy(zeros_hbm, acc_ref.at[pl.ds(sid * NPT, NPT)])
        pltpu.sync_copy(M, mv)
        msum = mv[0, :] + mv[1, :]
        cvec = jnp.where(msum > 0, msum, 0.2 * msum)
        plsc.subcore_barrier()

        def _off(j):
            return (wid + j * NW) * CHUNK

        def _fire_idx(j):
            pltpu.async_copy(S1.at[pl.ds(_off(j), CHUNK)], eibs.at[j % 3], semi)
            pltpu.async_copy(D1.at[pl.ds(_off(j), CHUNK)], eibd.at[j % 3], semi)

        def _wait_idx(j):
            pltpu.make_async_copy(S1.at[pl.ds(_off(j), CHUNK)], eibs.at[j % 3],
                                  semi).wait()
            pltpu.make_async_copy(D1.at[pl.ds(_off(j), CHUNK)], eibd.at[j % 3],
                                  semi).wait()

        def _fire_gather(j, b):
            pltpu.async_copy(T.at[eibs.at[j % 3]], g1.at[b], semg)
            pltpu.async_copy(R.at[eibd.at[j % 3]], g2.at[b], semg)

        def _wait_gather(j, b):
            pltpu.make_async_copy(T.at[eibs.at[j % 3]], g1.at[b], semg).wait()
            pltpu.make_async_copy(R.at[eibd.at[j % 3]], g2.at[b], semg).wait()

        _fire_idx(0)
        _fire_idx(1)
        _wait_idx(0)
        _fire_gather(0, 0)

        def chunk_body(j, _):
            b = j % 2
            _wait_gather(j, b)

            @pl.when(j + 2 < NLOC)
            def _():
                _fire_idx(j + 2)

            @pl.when(j + 1 < NLOC)
            def _():
                _wait_idx(j + 1)
                _fire_gather(j + 1, 1 - b)

            @pl.when(j >= 1)
            def _():
                pltpu.make_async_copy(
                    ob, acc_ref.at[eibd.at[(j - 1) % 3]], sems).wait()

            @plsc.parallel_loop(0, CHUNK, unroll=4)
            def _(i):
                el = g1[b, i, pl.ds(HD, L)]
                er = g2[b, i, :]
                sv = el + er
                ev = jnp.where(sv > 0, sv, 0.2 * sv)
                ee = jnp.exp(ev - cvec)
                ob[i, pl.ds(HD, L)] = ee
                for h in range(H):
                    ob[i, pl.ds(h * D, D)] = g1[b, i, pl.ds(h * D, D)] * ee[h]

            pltpu.async_copy(ob, acc_ref.at[eibd.at[j % 3]], sems, add=True)
            return 0

        lax.fori_loop(0, NLOC, chunk_body, 0)
        pltpu.make_async_copy(ob, acc_ref.at[eibd.at[(NLOC - 1) % 3]],
                              sems).wait()
        plsc.subcore_barrier()
        pltpu.sync_copy(
            acc_ref.at[pl.ds(sid * NPT, NPT)],
            out_hbm.at[p, cid, pl.ds(sid * NPT, NPT)])
        plsc.subcore_barrier()


def _make_sc_call():
    mesh = plsc.VectorSubcoreMesh(core_axis_name="c", subcore_axis_name="s",
                                  num_cores=NC, num_subcores=NS)

    return pl.kernel(
        _sc_body,
        out_type=jax.ShapeDtypeStruct((2, NC, NPAD, ROW), jnp.float32),
        mesh=mesh,
        scratch_types=[
            pltpu.VMEM((3, CHUNK), jnp.int32),
            pltpu.VMEM((3, CHUNK), jnp.int32),
            pltpu.VMEM((2, CHUNK, ROW), jnp.float32),
            pltpu.VMEM((2, CHUNK, L), jnp.float32),
            pltpu.VMEM((CHUNK, ROW), jnp.float32),
            pltpu.VMEM((2, L), jnp.float32),
            pltpu.VMEM_SHARED((NPAD, ROW), jnp.float32),
            pltpu.SemaphoreType.DMA,
            pltpu.SemaphoreType.DMA,
            pltpu.SemaphoreType.DMA,
        ],
        compiler_params=pltpu.CompilerParams(use_tc_tiling_on_sc=False),
    )


_sc_call = _make_sc_call()


def _finish_body(p_ref, b_ref, k_ref, sw_ref, sb_ref, sv_ref,
                 z0_ref, z1_ref, w_ref):
    i = pl.program_id(0)
    wps = []
    for pi, zref in ((0, z0_ref), (1, z1_ref)):
        agg = p_ref[pi, 0] + p_ref[pi, 1]
        den = jnp.dot(agg[:, HD:ROW], k_ref[...],
                      preferred_element_type=jnp.float32, precision=_HIGH)
        q = agg[:, :HD] / jnp.maximum(den, 1e-9) + b_ref[pi]
        z = jnp.where(q > 0, q, jnp.exp(jnp.minimum(q, 0.0)) - 1.0)
        zref[...] = z
        t = jnp.tanh(jnp.dot(z, sw_ref[...], preferred_element_type=jnp.float32,
                             precision=_HIGH) + sb_ref[...])
        wps.append(jnp.full((1, 1, HD), jnp.sum(t * sv_ref[...])))
    wp = jnp.concatenate(wps, axis=0)

    @pl.when(i == 0)
    def _():
        w_ref[...] = wp

    @pl.when(i != 0)
    def _():
        w_ref[...] = w_ref[...] + wp


_finish_call = pl.pallas_call(
    _finish_body,
    grid=(NB,),
    in_specs=[
        pl.BlockSpec((2, NC, BN, ROW), lambda i: (0, 0, i, 0)),
        pl.BlockSpec((2, 1, HD), lambda i: (0, 0, 0)),
        pl.BlockSpec((L, HD), lambda i: (0, 0)),
        pl.BlockSpec((HD, SEM), lambda i: (0, 0)),
        pl.BlockSpec((1, SEM), lambda i: (0, 0)),
        pl.BlockSpec((1, SEM), lambda i: (0, 0)),
    ],
    out_specs=[
        pl.BlockSpec((BN, HD), lambda i: (i, 0)),
        pl.BlockSpec((BN, HD), lambda i: (i, 0)),
        pl.BlockSpec((2, 1, HD), lambda i: (0, 0, 0)),
    ],
    out_shape=[
        jax.ShapeDtypeStruct((N, HD), jnp.float32),
        jax.ShapeDtypeStruct((N, HD), jnp.float32),
        jax.ShapeDtypeStruct((2, 1, HD), jnp.float32),
    ],
)


def _combine_body(z0_ref, z1_ref, w_ref, o_ref):
    w0 = w_ref[0, 0, 0] / np.float32(N)
    w1 = w_ref[1, 0, 0] / np.float32(N)
    m = jnp.maximum(w0, w1)
    e0 = jnp.exp(w0 - m)
    e1 = jnp.exp(w1 - m)
    b0 = e0 / (e0 + e1)
    b1 = e1 / (e0 + e1)
    o_ref[...] = b0 * z0_ref[...] + b1 * z1_ref[...]


_combine_call = pl.pallas_call(
    _combine_body,
    grid=(NB,),
    in_specs=[
        pl.BlockSpec((BN, HD), lambda i: (i, 0)),
        pl.BlockSpec((BN, HD), lambda i: (i, 0)),
        pl.BlockSpec((2, 1, HD), lambda i: (0, 0, 0)),
    ],
    out_specs=pl.BlockSpec((BN, HD), lambda i: (i, 0)),
    out_shape=jax.ShapeDtypeStruct((N, HD), jnp.float32),
)


def _pack_weights(W, al, ar):
    sel = jnp.asarray(np.kron(np.eye(H), np.ones((D, 1))), jnp.float32)
    Wl = (W * al.reshape(-1)[None, :]) @ sel
    Wr = (W * ar.reshape(-1)[None, :]) @ sel
    Wt = jnp.concatenate([W, Wl, Wl], axis=1)
    Wr2 = jnp.concatenate([Wr, Wr], axis=1)
    return Wt, Wr2


def kernel(x, edge_index_0, edge_index_1, W_gat0, b_gat0, attn_l0, attn_r0,
           W_gat1, b_gat1, attn_l1, attn_r1, sem_W, sem_b, sem_v):
    Wt0, Wr20 = _pack_weights(W_gat0, attn_l0, attn_r0)
    Wt1, Wr21 = _pack_weights(W_gat1, attn_l1, attn_r1)
    T0, R0, m0 = _prep_call(x, Wt0, Wr20)
    T1, R1, m1 = _prep_call(x, Wt1, Wr21)
    zeros = jnp.zeros((NPT, ROW), jnp.float32)
    P = _sc_call(T0, R0, edge_index_0[0], edge_index_0[1], m0,
                 T1, R1, edge_index_1[0], edge_index_1[1], m1, zeros)
    ksel = jnp.asarray(np.kron(np.eye(H), np.ones((1, D))), jnp.float32)
    K = jnp.concatenate([ksel, jnp.zeros((H, HD), jnp.float32)], axis=0)
    bstack = jnp.stack([b_gat0, b_gat1]).reshape(2, 1, HD)
    z0, z1, wsum = _finish_call(P, bstack, K, sem_W, sem_b.reshape(1, SEM),
                                sem_v.reshape(SEM, 1).T)
    return _combine_call(z0, z1, wsum)


# speedup vs baseline: 1.8277x; 1.0010x over previous
import jax
import jax.numpy as jnp
import numpy as np
from jax import lax
from jax.experimental import pallas as pl
from jax.experimental.pallas import tpu as pltpu
from jax.experimental.pallas import tpu_sc as plsc

N = 10000
E = 320000
H = 8
D = 16
HD = H * D
ROW = HD + 16
SEM = 128
NC, NS, L = 2, 16, 16
NW = NC * NS
CHUNK = 80
NCHUNK = E // CHUNK
NLOC = NCHUNK // NW
NPAD = 10112
NPT = NPAD // NS
BN = 1000
NB = N // BN

_HIGH = lax.Precision.HIGHEST


def _prep_body(x_ref, wt_ref, wr_ref, t_ref, r_ref, m_ref):
    i = pl.program_id(0)
    xb = x_ref[...]
    t = jnp.dot(xb, wt_ref[...], preferred_element_type=jnp.float32,
                precision=_HIGH)
    r = jnp.dot(xb, wr_ref[...], preferred_element_type=jnp.float32,
                precision=_HIGH)
    t_ref[...] = t
    r_ref[...] = r
    elmax = jnp.max(t[:, HD:ROW], axis=0, keepdims=True)
    ermax = jnp.max(r, axis=0, keepdims=True)
    cur = jnp.concatenate([elmax, ermax], axis=0)

    @pl.when(i == 0)
    def _():
        m_ref[...] = cur

    @pl.when(i != 0)
    def _():
        m_ref[...] = jnp.maximum(m_ref[...], cur)


_prep_call = pl.pallas_call(
    _prep_body,
    grid=(NB,),
    in_specs=[
        pl.BlockSpec((BN, HD), lambda i: (i, 0)),
        pl.BlockSpec((HD, ROW), lambda i: (0, 0)),
        pl.BlockSpec((HD, L), lambda i: (0, 0)),
    ],
    out_specs=[
        pl.BlockSpec((BN, ROW), lambda i: (i, 0)),
        pl.BlockSpec((BN, L), lambda i: (i, 0)),
        pl.BlockSpec((2, L), lambda i: (0, 0)),
    ],
    out_shape=[
        jax.ShapeDtypeStruct((N, ROW), jnp.float32),
        jax.ShapeDtypeStruct((N, L), jnp.float32),
        jax.ShapeDtypeStruct((2, L), jnp.float32),
    ],
)


def _sc_body(t0, r0, s0, d0, m0, t1, r1, s1, d1, m1, zeros_hbm, out_hbm,
             eibs, eibd, g1, g2, ob, mv, acc_ref, semi, semg, sems):
    cid = lax.axis_index("c")
    sid = lax.axis_index("s")
    wid = sid * NC + cid

    for p, (T, R, S1, D1, M) in enumerate(((t0, r0, s0, d0, m0),
                                           (t1, r1, s1, d1, m1))):
        pltpu.sync_copy(zeros_hbm, acc_ref.at[pl.ds(sid * NPT, NPT)])
        pltpu.sync_copy(M, mv)
        msum = mv[0, :] + mv[1, :]
        cvec = jnp.where(msum > 0, msum, 0.2 * msum)
        plsc.subcore_barrier()

        def _off(j):
            return (wid + j * NW) * CHUNK

        def _fire_idx(j):
            pltpu.async_copy(S1.at[pl.ds(_off(j), CHUNK)], eibs.at[j % 3], semi)
            pltpu.async_copy(D1.at[pl.ds(_off(j), CHUNK)], eibd.at[j % 3], semi)

        def _wait_idx(j):
            pltpu.make_async_copy(S1.at[pl.ds(_off(j), CHUNK)], eibs.at[j % 3],
                                  semi).wait()
            pltpu.make_async_copy(D1.at[pl.ds(_off(j), CHUNK)], eibd.at[j % 3],
                                  semi).wait()

        def _fire_gather(j, b):
            pltpu.async_copy(T.at[eibs.at[j % 3]], g1.at[b], semg)
            pltpu.async_copy(R.at[eibd.at[j % 3]], g2.at[b], semg)

        def _wait_gather(j, b):
            pltpu.make_async_copy(T.at[eibs.at[j % 3]], g1.at[b], semg).wait()
            pltpu.make_async_copy(R.at[eibd.at[j % 3]], g2.at[b], semg).wait()

        _fire_idx(0)
        _fire_idx(1)
        _wait_idx(0)
        _fire_gather(0, 0)

        def chunk_body(j, _):
            b = j % 2
            _wait_gather(j, b)

            @pl.when(j + 2 < NLOC)
            def _():
                _fire_idx(j + 2)

            @pl.when(j + 1 < NLOC)
            def _():
                _wait_idx(j + 1)
                _fire_gather(j + 1, 1 - b)

            @pl.when(j >= 1)
            def _():
                pltpu.make_async_copy(
                    ob, acc_ref.at[eibd.at[(j - 1) % 3]], sems).wait()

            @plsc.parallel_loop(0, CHUNK, unroll=4)
            def _(i):
                el = g1[b, i, pl.ds(HD, L)]
                er = g2[b, i, :]
                sv = el + er
                ev = jnp.where(sv > 0, sv, 0.2 * sv)
                ee = jnp.exp(ev - cvec)
                ob[i, pl.ds(HD, L)] = ee
                for h in range(H):
                    ob[i, pl.ds(h * D, D)] = g1[b, i, pl.ds(h * D, D)] * ee[h]

            pltpu.async_copy(ob, acc_ref.at[eibd.at[j % 3]], sems, add=True)
            return 0

        lax.fori_loop(0, NLOC, chunk_body, 0)
        pltpu.make_async_copy(ob, acc_ref.at[eibd.at[(NLOC - 1) % 3]],
                              sems).wait()
        plsc.subcore_barrier()
        pltpu.sync_copy(
            acc_ref.at[pl.ds(sid * NPT, NPT)],
            out_hbm.at[p, cid, pl.ds(sid * NPT, NPT)])
        plsc.subcore_barrier()


def _make_sc_call():
    mesh = plsc.VectorSubcoreMesh(core_axis_name="c", subcore_axis_name="s",
                                  num_cores=NC, num_subcores=NS)

    return pl.kernel(
        _sc_body,
        out_type=jax.ShapeDtypeStruct((2, NC, NPAD, ROW), jnp.float32),
        mesh=mesh,
        scratch_types=[
            pltpu.VMEM((3, CHUNK), jnp.int32),
            pltpu.VMEM((3, CHUNK), jnp.int32),
            pltpu.VMEM((2, CHUNK, ROW), jnp.float32),
            pltpu.VMEM((2, CHUNK, L), jnp.float32),
            pltpu.VMEM((CHUNK, ROW), jnp.float32),
            pltpu.VMEM((2, L), jnp.float32),
            pltpu.VMEM_SHARED((NPAD, ROW), jnp.float32),
            pltpu.SemaphoreType.DMA,
            pltpu.SemaphoreType.DMA,
            pltpu.SemaphoreType.DMA,
        ],
        compiler_params=pltpu.CompilerParams(use_tc_tiling_on_sc=False),
    )


_sc_call = _make_sc_call()


def _finish_body(p_ref, b_ref, k_ref, sw_ref, sb_ref, sv_ref,
                 z0_ref, z1_ref, w_ref):
    i = pl.program_id(0)
    wps = []
    for pi, zref in ((0, z0_ref), (1, z1_ref)):
        agg = p_ref[pi, 0] + p_ref[pi, 1]
        den = jnp.dot(agg[:, HD:ROW], k_ref[...],
                      preferred_element_type=jnp.float32, precision=_HIGH)
        q = agg[:, :HD] / jnp.maximum(den, 1e-9) + b_ref[pi]
        z = jnp.where(q > 0, q, jnp.exp(jnp.minimum(q, 0.0)) - 1.0)
        zref[...] = z
        t = jnp.tanh(jnp.dot(z, sw_ref[...], preferred_element_type=jnp.float32,
                             precision=_HIGH) + sb_ref[...])
        wps.append(jnp.full((1, 1, HD), jnp.sum(t * sv_ref[...])))
    wp = jnp.concatenate(wps, axis=0)

    @pl.when(i == 0)
    def _():
        w_ref[...] = wp

    @pl.when(i != 0)
    def _():
        w_ref[...] = w_ref[...] + wp


_finish_call = pl.pallas_call(
    _finish_body,
    grid=(NB,),
    in_specs=[
        pl.BlockSpec((2, NC, BN, ROW), lambda i: (0, 0, i, 0)),
        pl.BlockSpec((2, 1, HD), lambda i: (0, 0, 0)),
        pl.BlockSpec((L, HD), lambda i: (0, 0)),
        pl.BlockSpec((HD, SEM), lambda i: (0, 0)),
        pl.BlockSpec((1, SEM), lambda i: (0, 0)),
        pl.BlockSpec((1, SEM), lambda i: (0, 0)),
    ],
    out_specs=[
        pl.BlockSpec((BN, HD), lambda i: (i, 0)),
        pl.BlockSpec((BN, HD), lambda i: (i, 0)),
        pl.BlockSpec((2, 1, HD), lambda i: (0, 0, 0)),
    ],
    out_shape=[
        jax.ShapeDtypeStruct((N, HD), jnp.float32),
        jax.ShapeDtypeStruct((N, HD), jnp.float32),
        jax.ShapeDtypeStruct((2, 1, HD), jnp.float32),
    ],
)


def _combine_body(z0_ref, z1_ref, w_ref, o_ref):
    w0 = w_ref[0, 0, 0] / np.float32(N)
    w1 = w_ref[1, 0, 0] / np.float32(N)
    m = jnp.maximum(w0, w1)
    e0 = jnp.exp(w0 - m)
    e1 = jnp.exp(w1 - m)
    b0 = e0 / (e0 + e1)
    b1 = e1 / (e0 + e1)
    o_ref[...] = b0 * z0_ref[...] + b1 * z1_ref[...]


_combine_call = pl.pallas_call(
    _combine_body,
    grid=(NB,),
    in_specs=[
        pl.BlockSpec((BN, HD), lambda i: (i, 0)),
        pl.BlockSpec((BN, HD), lambda i: (i, 0)),
        pl.BlockSpec((2, 1, HD), lambda i: (0, 0, 0)),
    ],
    out_specs=pl.BlockSpec((BN, HD), lambda i: (i, 0)),
    out_shape=jax.ShapeDtypeStruct((N, HD), jnp.float32),
)


def _pack_weights(W, al, ar):
    sel = jnp.asarray(np.kron(np.eye(H), np.ones((D, 1))), jnp.float32)
    Wl = (W * al.reshape(-1)[None, :]) @ sel
    Wr = (W * ar.reshape(-1)[None, :]) @ sel
    Wt = jnp.concatenate([W, Wl, Wl], axis=1)
    Wr2 = jnp.concatenate([Wr, Wr], axis=1)
    return Wt, Wr2


def kernel(x, edge_index_0, edge_index_1, W_gat0, b_gat0, attn_l0, attn_r0,
           W_gat1, b_gat1, attn_l1, attn_r1, sem_W, sem_b, sem_v):
    Wt0, Wr20 = _pack_weights(W_gat0, attn_l0, attn_r0)
    Wt1, Wr21 = _pack_weights(W_gat1, attn_l1, attn_r1)
    T0, R0, m0 = _prep_call(x, Wt0, Wr20)
    T1, R1, m1 = _prep_call(x, Wt1, Wr21)
    zeros = jnp.zeros((NPT, ROW), jnp.float32)
    P = _sc_call(T0, R0, edge_index_0[0], edge_index_0[1], m0,
                 T1, R1, edge_index_1[0], edge_index_1[1], m1, zeros)
    ksel = jnp.asarray(np.kron(np.eye(H), np.ones((1, D))), jnp.float32)
    K = jnp.concatenate([ksel, jnp.zeros((H, HD), jnp.float32)], axis=0)
    bstack = jnp.stack([b_gat0, b_gat1]).reshape(2, 1, HD)
    z0, z1, wsum = _finish_call(P, bstack, K, sem_W, sem_b.reshape(1, SEM),
                                sem_v.reshape(SEM, 1).T)
    return _combine_call(z0, z1, wsum)
